# Initial kernel scaffold; baseline (speedup 1.0000x reference)
import jax
import jax.numpy as jnp
from jax.experimental import pallas as pl


def kernel(points, queries, feat_prev, W1, b1, g1, be1, W2, b2, g2, be2):
    raise NotImplementedError("write your pallas kernel here")


# speedup vs baseline: 3.4709x; 3.4709x over previous
import jax
import jax.numpy as jnp
from jax.experimental import pallas as pl

K = 32
MB = 128
RB = 8192


def _knn_kernel(q_ref, pt_ref, p_ref, loc_ref):
    q = q_ref[0]
    pt = pt_ref[0]
    p = p_ref[0]
    n = pt.shape[1]

    pq = jnp.dot(q, pt, preferred_element_type=jnp.float32)
    pn = ((pt[0:1] * pt[0:1] + pt[1:2] * pt[1:2])
          + pt[2:3] * pt[2:3])
    qn = jnp.sum(q * q, axis=1)[:, None]
    d = jnp.sqrt(jnp.maximum(pn + qn - 2.0 * pq, 0.0))

    iota = jax.lax.broadcasted_iota(jnp.int32, (MB, n), 1)
    inf = jnp.float32(jnp.inf)

    ph = p.astype(jnp.bfloat16).astype(jnp.float32)
    r = p - ph
    pm = r.astype(jnp.bfloat16).astype(jnp.float32)
    plo = r - pm

    locs = []
    for _ in range(K):
        mval = jnp.min(d, axis=1)
        is_min = d == mval[:, None]
        idx = jnp.min(jnp.where(is_min, iota, n), axis=1)
        oh = iota == idx[:, None]
        ohf = oh.astype(jnp.float32)
        coords = (jnp.dot(ohf, ph, preferred_element_type=jnp.float32)
                  + jnp.dot(ohf, pm, preferred_element_type=jnp.float32)) \
            + jnp.dot(ohf, plo, preferred_element_type=jnp.float32)
        locs.append(q - coords)
        d = jnp.where(oh, inf, d)

    loc_ref[0] = jnp.concatenate(locs, axis=1)


def _mlp1_kernel(l_ref, w_ref, b_ref, h_ref, mom_ref):
    i = pl.program_id(0)
    h = jnp.dot(l_ref[...], w_ref[...],
                preferred_element_type=jnp.float32) + b_ref[...]
    h_ref[...] = h
    s1 = jnp.sum(h, axis=0)
    s2 = jnp.sum(h * h, axis=0)
    blk = jnp.concatenate([
        jnp.pad(s1[None, :], ((0, 0), (0, 96))),
        jnp.pad(s2[None, :], ((0, 0), (0, 96))),
        jnp.zeros((6, 128), jnp.float32),
    ], axis=0)

    @pl.when(i == 0)
    def _():
        mom_ref[...] = blk

    @pl.when(i != 0)
    def _():
        mom_ref[...] += blk


def _mlp2_kernel(h_ref, m_ref, s_ref, g_ref, be_ref, w_ref, b_ref,
                 h2_ref, mom_ref):
    i = pl.program_id(0)
    a = ((h_ref[...] - m_ref[...]) * s_ref[...]) * g_ref[...] + be_ref[...]
    a = jnp.maximum(a, 0.0)
    h2 = jnp.dot(a, w_ref[...],
                 preferred_element_type=jnp.float32) + b_ref[...]
    h2_ref[...] = h2
    s1 = jnp.sum(h2, axis=0)
    s2 = jnp.sum(h2 * h2, axis=0)
    blk = jnp.concatenate([
        jnp.pad(s1[None, :], ((0, 0), (0, 64))),
        jnp.pad(s2[None, :], ((0, 0), (0, 64))),
        jnp.zeros((6, 128), jnp.float32),
    ], axis=0)

    @pl.when(i == 0)
    def _():
        mom_ref[...] = blk

    @pl.when(i != 0)
    def _():
        mom_ref[...] += blk


def _bn_kernel(h_ref, m_ref, s_ref, g_ref, be_ref, o_ref):
    o = ((h_ref[...] - m_ref[...]) * s_ref[...]) * g_ref[...] + be_ref[...]
    o_ref[...] = jnp.maximum(o, 0.0)


def kernel(points, queries, feat_prev, W1, b1, g1, be1, W2, b2, g2, be2):
    B, L, N, _ = points.shape
    M = queries.shape[2]
    p = points.reshape(B, N, 3)
    q = queries.reshape(B, M, 3)
    pt = jnp.swapaxes(p, 1, 2)
    nt = B * L * M * K

    loc = pl.pallas_call(
        _knn_kernel,
        grid=(B, M // MB),
        in_specs=[
            pl.BlockSpec((1, MB, 3), lambda b, mi: (b, mi, 0)),
            pl.BlockSpec((1, 3, N), lambda b, mi: (b, 0, 0)),
            pl.BlockSpec((1, N, 3), lambda b, mi: (b, 0, 0)),
        ],
        out_specs=pl.BlockSpec((1, MB, K * 3), lambda b, mi: (b, mi, 0)),
        out_shape=jax.ShapeDtypeStruct((B, M, K * 3), jnp.float32),
    )(q, pt, p)

    lrows = loc.reshape(nt, 3)
    h1, mom1 = pl.pallas_call(
        _mlp1_kernel,
        grid=(nt // RB,),
        in_specs=[
            pl.BlockSpec((RB, 3), lambda i: (i, 0)),
            pl.BlockSpec((3, 32), lambda i: (0, 0)),
            pl.BlockSpec((1, 32), lambda i: (0, 0)),
        ],
        out_specs=[
            pl.BlockSpec((RB, 32), lambda i: (i, 0)),
            pl.BlockSpec((8, 128), lambda i: (0, 0)),
        ],
        out_shape=[
            jax.ShapeDtypeStruct((nt, 32), jnp.float32),
            jax.ShapeDtypeStruct((8, 128), jnp.float32),
        ],
    )(lrows, W1, b1[None, :])

    mean1 = (mom1[0, 0:32] / nt)[None, :]
    var1 = mom1[1, 0:32] / nt - mean1[0] * mean1[0]
    i1 = (1.0 / jnp.sqrt(var1 + 1e-5))[None, :]

    h2, mom2 = pl.pallas_call(
        _mlp2_kernel,
        grid=(nt // RB,),
        in_specs=[
            pl.BlockSpec((RB, 32), lambda i: (i, 0)),
            pl.BlockSpec((1, 32), lambda i: (0, 0)),
            pl.BlockSpec((1, 32), lambda i: (0, 0)),
            pl.BlockSpec((1, 32), lambda i: (0, 0)),
            pl.BlockSpec((1, 32), lambda i: (0, 0)),
            pl.BlockSpec((32, 64), lambda i: (0, 0)),
            pl.BlockSpec((1, 64), lambda i: (0, 0)),
        ],
        out_specs=[
            pl.BlockSpec((RB, 64), lambda i: (i, 0)),
            pl.BlockSpec((8, 128), lambda i: (0, 0)),
        ],
        out_shape=[
            jax.ShapeDtypeStruct((nt, 64), jnp.float32),
            jax.ShapeDtypeStruct((8, 128), jnp.float32),
        ],
    )(h1, mean1, i1, g1[None, :], be1[None, :], W2, b2[None, :])

    mean2 = (mom2[0, 0:64] / nt)[None, :]
    var2 = mom2[1, 0:64] / nt - mean2[0] * mean2[0]
    i2 = (1.0 / jnp.sqrt(var2 + 1e-5))[None, :]

    out = pl.pallas_call(
        _bn_kernel,
        grid=(nt // RB,),
        in_specs=[
            pl.BlockSpec((RB, 64), lambda i: (i, 0)),
            pl.BlockSpec((1, 64), lambda i: (0, 0)),
            pl.BlockSpec((1, 64), lambda i: (0, 0)),
            pl.BlockSpec((1, 64), lambda i: (0, 0)),
            pl.BlockSpec((1, 64), lambda i: (0, 0)),
        ],
        out_specs=pl.BlockSpec((RB, 64), lambda i: (i, 0)),
        out_shape=jax.ShapeDtypeStruct((nt, 64), jnp.float32),
    )(h2, mean2, i2, g2[None, :], be2[None, :])

    return out.reshape(B, L, M, K, 64)


# speedup vs baseline: 3.9951x; 1.1510x over previous
import jax
import jax.numpy as jnp
from jax.experimental import pallas as pl

K = 32
MB = 256
RB = 8192


def _knn_kernel(q_ref, pt_ref, p_ref, loc_ref):
    q = q_ref[0]
    pt = pt_ref[0]
    p = p_ref[0]
    n = pt.shape[1]

    pq = jnp.dot(q, pt, preferred_element_type=jnp.float32)
    pn = ((pt[0:1] * pt[0:1] + pt[1:2] * pt[1:2])
          + pt[2:3] * pt[2:3])
    qn = jnp.sum(q * q, axis=1)[:, None]
    d = jnp.sqrt(jnp.maximum(pn + qn - 2.0 * pq, 0.0))

    iota = jax.lax.broadcasted_iota(jnp.int32, (MB, n), 1)
    inf = jnp.float32(jnp.inf)

    ph = p.astype(jnp.bfloat16).astype(jnp.float32)
    r = p - ph
    pm = r.astype(jnp.bfloat16).astype(jnp.float32)
    plo = r - pm

    locs = []
    for _ in range(K):
        mval = jnp.min(d, axis=1)
        is_min = d == mval[:, None]
        idx = jnp.min(jnp.where(is_min, iota, n), axis=1)
        oh = iota == idx[:, None]
        ohf = oh.astype(jnp.float32)
        coords = (jnp.dot(ohf, ph, preferred_element_type=jnp.float32)
                  + jnp.dot(ohf, pm, preferred_element_type=jnp.float32)) \
            + jnp.dot(ohf, plo, preferred_element_type=jnp.float32)
        locs.append(q - coords)
        d = jnp.where(oh, inf, d)

    loc_ref[0] = jnp.concatenate(locs, axis=1)


def _mlp1_kernel(l_ref, w_ref, b_ref, h_ref, mom_ref):
    i = pl.program_id(0)
    h = jnp.dot(l_ref[...], w_ref[...],
                preferred_element_type=jnp.float32) + b_ref[...]
    h_ref[...] = h
    s1 = jnp.sum(h, axis=0)
    s2 = jnp.sum(h * h, axis=0)
    blk = jnp.concatenate([
        jnp.pad(s1[None, :], ((0, 0), (0, 96))),
        jnp.pad(s2[None, :], ((0, 0), (0, 96))),
        jnp.zeros((6, 128), jnp.float32),
    ], axis=0)

    @pl.when(i == 0)
    def _():
        mom_ref[...] = blk

    @pl.when(i != 0)
    def _():
        mom_ref[...] += blk


def _mlp2_kernel(h_ref, m_ref, s_ref, g_ref, be_ref, w_ref, b_ref,
                 h2_ref, mom_ref):
    i = pl.program_id(0)
    a = ((h_ref[...] - m_ref[...]) * s_ref[...]) * g_ref[...] + be_ref[...]
    a = jnp.maximum(a, 0.0)
    h2 = jnp.dot(a, w_ref[...],
                 preferred_element_type=jnp.float32) + b_ref[...]
    h2_ref[...] = h2
    s1 = jnp.sum(h2, axis=0)
    s2 = jnp.sum(h2 * h2, axis=0)
    blk = jnp.concatenate([
        jnp.pad(s1[None, :], ((0, 0), (0, 64))),
        jnp.pad(s2[None, :], ((0, 0), (0, 64))),
        jnp.zeros((6, 128), jnp.float32),
    ], axis=0)

    @pl.when(i == 0)
    def _():
        mom_ref[...] = blk

    @pl.when(i != 0)
    def _():
        mom_ref[...] += blk


def _bn_kernel(h_ref, m_ref, s_ref, g_ref, be_ref, o_ref):
    o = ((h_ref[...] - m_ref[...]) * s_ref[...]) * g_ref[...] + be_ref[...]
    o_ref[...] = jnp.maximum(o, 0.0)


def kernel(points, queries, feat_prev, W1, b1, g1, be1, W2, b2, g2, be2):
    B, L, N, _ = points.shape
    M = queries.shape[2]
    p = points.reshape(B, N, 3)
    q = queries.reshape(B, M, 3)
    pt = jnp.swapaxes(p, 1, 2)
    nt = B * L * M * K

    loc = pl.pallas_call(
        _knn_kernel,
        grid=(B, M // MB),
        in_specs=[
            pl.BlockSpec((1, MB, 3), lambda b, mi: (b, mi, 0)),
            pl.BlockSpec((1, 3, N), lambda b, mi: (b, 0, 0)),
            pl.BlockSpec((1, N, 3), lambda b, mi: (b, 0, 0)),
        ],
        out_specs=pl.BlockSpec((1, MB, K * 3), lambda b, mi: (b, mi, 0)),
        out_shape=jax.ShapeDtypeStruct((B, M, K * 3), jnp.float32),
    )(q, pt, p)

    lrows = loc.reshape(nt, 3)
    h1, mom1 = pl.pallas_call(
        _mlp1_kernel,
        grid=(nt // RB,),
        in_specs=[
            pl.BlockSpec((RB, 3), lambda i: (i, 0)),
            pl.BlockSpec((3, 32), lambda i: (0, 0)),
            pl.BlockSpec((1, 32), lambda i: (0, 0)),
        ],
        out_specs=[
            pl.BlockSpec((RB, 32), lambda i: (i, 0)),
            pl.BlockSpec((8, 128), lambda i: (0, 0)),
        ],
        out_shape=[
            jax.ShapeDtypeStruct((nt, 32), jnp.float32),
            jax.ShapeDtypeStruct((8, 128), jnp.float32),
        ],
    )(lrows, W1, b1[None, :])

    mean1 = (mom1[0, 0:32] / nt)[None, :]
    var1 = mom1[1, 0:32] / nt - mean1[0] * mean1[0]
    i1 = (1.0 / jnp.sqrt(var1 + 1e-5))[None, :]

    h2, mom2 = pl.pallas_call(
        _mlp2_kernel,
        grid=(nt // RB,),
        in_specs=[
            pl.BlockSpec((RB, 32), lambda i: (i, 0)),
            pl.BlockSpec((1, 32), lambda i: (0, 0)),
            pl.BlockSpec((1, 32), lambda i: (0, 0)),
            pl.BlockSpec((1, 32), lambda i: (0, 0)),
            pl.BlockSpec((1, 32), lambda i: (0, 0)),
            pl.BlockSpec((32, 64), lambda i: (0, 0)),
            pl.BlockSpec((1, 64), lambda i: (0, 0)),
        ],
        out_specs=[
            pl.BlockSpec((RB, 64), lambda i: (i, 0)),
            pl.BlockSpec((8, 128), lambda i: (0, 0)),
        ],
        out_shape=[
            jax.ShapeDtypeStruct((nt, 64), jnp.float32),
            jax.ShapeDtypeStruct((8, 128), jnp.float32),
        ],
    )(h1, mean1, i1, g1[None, :], be1[None, :], W2, b2[None, :])

    mean2 = (mom2[0, 0:64] / nt)[None, :]
    var2 = mom2[1, 0:64] / nt - mean2[0] * mean2[0]
    i2 = (1.0 / jnp.sqrt(var2 + 1e-5))[None, :]

    out = pl.pallas_call(
        _bn_kernel,
        grid=(nt // RB,),
        in_specs=[
            pl.BlockSpec((RB, 64), lambda i: (i, 0)),
            pl.BlockSpec((1, 64), lambda i: (0, 0)),
            pl.BlockSpec((1, 64), lambda i: (0, 0)),
            pl.BlockSpec((1, 64), lambda i: (0, 0)),
            pl.BlockSpec((1, 64), lambda i: (0, 0)),
        ],
        out_specs=pl.BlockSpec((RB, 64), lambda i: (i, 0)),
        out_shape=jax.ShapeDtypeStruct((nt, 64), jnp.float32),
    )(h2, mean2, i2, g2[None, :], be2[None, :])

    return out.reshape(B, L, M, K, 64)
